# baseline (speedup 1.0000x reference)
import jax
import jax.numpy as jnp
from jax.experimental import pallas as pl
from jax.experimental.pallas import tpu as pltpu

_N, _D, _H = 10000, 128, 16
_BM = 200
_NB = _N // _BM
_BC = 200
_NC = _N // _BC
_SLOTS = 4
_LOOK = 3


def _xwt_kernel(x_ref, w_ref, out_ref):
    out_ref[...] = jnp.dot(x_ref[...], w_ref[...],
                           preferred_element_type=jnp.float32).T


def _adj_copy(adj_hbm, abuf, sems, blk, slot):
    return pltpu.make_async_copy(
        adj_hbm.at[pl.ds(blk * _BM, _BM), :], abuf.at[slot], sems.at[slot])


def _mega_kernel(adj_hbm, xwt_ref, wfc_ref, aff_ref,
                 mu_ref, lv_ref, xrec_ref, rec_ref,
                 abuf, z_s, sems):
    s = pl.program_id(0)

    @pl.when(s == 0)
    def _prologue():
        for j in range(_LOOK):
            _adj_copy(adj_hbm, abuf, sems, j, j).start()

    @pl.when(s < _NB)
    def _gc_phase():
        nxt = s + _LOOK

        @pl.when(nxt < _NB)
        def _prefetch():
            _adj_copy(adj_hbm, abuf, sems, nxt, nxt % _SLOTS).start()

        _adj_copy(adj_hbm, abuf, sems, s, s % _SLOTS).wait()
        t = jax.lax.dot_general(
            abuf[s % _SLOTS], xwt_ref[...], (((1,), (1,)), ((), ())),
            preferred_element_type=jnp.float32)
        t = jnp.where(t >= 0, t, 0.01 * t)
        mu = t[:, :_H]
        mu_ref[...] = mu
        lv_ref[...] = t[:, _H:]
        z_s[pl.ds(s * _BM, _BM), :] = mu
        h = jnp.dot(mu, wfc_ref[...], preferred_element_type=jnp.float32)
        xrec_ref[...] = h * aff_ref[0:1, :] + aff_ref[1:2, :]

    @pl.when(s >= _NB)
    def _ip_phase():
        zb = z_s[pl.ds((s - _NB) * _BC, _BC), :]
        rec_ref[...] = jax.lax.dot_general(
            zb, z_s[...], (((1,), (1,)), ((), ())),
            preferred_element_type=jnp.float32)


def kernel(x, adj, W1, W2, Wfc, bfc, gamma, beta, running_mean, running_var):
    n, d = x.shape
    h = W1.shape[1]

    wcat = jnp.concatenate([W1, W2], axis=1)
    scale = gamma * jax.lax.rsqrt(running_var + 1e-5)
    shift = (bfc - running_mean) * scale + beta
    aff = jnp.stack([scale, shift], axis=0)

    xwt = pl.pallas_call(
        _xwt_kernel,
        grid=(1,),
        in_specs=[
            pl.BlockSpec((n, d), lambda i: (0, 0)),
            pl.BlockSpec((d, 2 * h), lambda i: (0, 0)),
        ],
        out_specs=pl.BlockSpec((2 * h, n), lambda i: (0, 0)),
        out_shape=jax.ShapeDtypeStruct((2 * h, n), jnp.float32),
    )(x, wcat)

    gc_idx = lambda s: (jnp.clip(s, 0, _NB - 1), 0)
    ip_idx = lambda s: (jnp.clip(s - _NB, 0, _NC - 1), 0)

    mu, logvar, x_rec, adj_rec = pl.pallas_call(
        _mega_kernel,
        grid=(_NB + _NC,),
        in_specs=[
            pl.BlockSpec(memory_space=pltpu.MemorySpace.HBM),
            pl.BlockSpec((2 * h, n), lambda s: (0, 0)),
            pl.BlockSpec((h, d), lambda s: (0, 0)),
            pl.BlockSpec((2, d), lambda s: (0, 0)),
        ],
        out_specs=[
            pl.BlockSpec((_BM, h), gc_idx),
            pl.BlockSpec((_BM, h), gc_idx),
            pl.BlockSpec((_BM, d), gc_idx),
            pl.BlockSpec((_BC, n), ip_idx),
        ],
        out_shape=[
            jax.ShapeDtypeStruct((n, h), jnp.float32),
            jax.ShapeDtypeStruct((n, h), jnp.float32),
            jax.ShapeDtypeStruct((n, d), jnp.float32),
            jax.ShapeDtypeStruct((n, n), jnp.float32),
        ],
        scratch_shapes=[
            pltpu.VMEM((_SLOTS, _BM, n), jnp.float32),
            pltpu.VMEM((n, h), jnp.float32),
            pltpu.SemaphoreType.DMA((_SLOTS,)),
        ],
    )(adj, xwt, Wfc, aff)

    z = mu
    return (adj_rec, mu, logvar, z, x_rec)


# speedup vs baseline: 1.0043x; 1.0043x over previous
import jax
import jax.numpy as jnp
from jax.experimental import pallas as pl
from jax.experimental.pallas import tpu as pltpu

_N, _D, _H = 10000, 128, 16
_BM = 200
_NB = _N // _BM
_BC = 200
_NC = _N // _BC


def _xwt_kernel(x_ref, w_ref, out_ref):
    out_ref[...] = jnp.dot(x_ref[...], w_ref[...],
                           preferred_element_type=jnp.float32).T


def _mega_kernel(adj_ref, xwt_ref, wfc_ref, aff_ref,
                 mu_ref, lv_ref, xrec_ref, rec_ref,
                 t_s, xrec_s):
    s = pl.program_id(0)

    @pl.when(s < _NB)
    def _gc_phase():
        t = jax.lax.dot_general(
            adj_ref[...], xwt_ref[...], (((1,), (1,)), ((), ())),
            preferred_element_type=jnp.float32)
        t = jnp.where(t >= 0, t, 0.01 * t)
        t_s[pl.ds(s * _BM, _BM), :] = t
        h = jnp.dot(t[:, :_H], wfc_ref[...],
                    preferred_element_type=jnp.float32)
        xrec_s[pl.ds(s * _BM, _BM), :] = (h * aff_ref[0:1, :]
                                          + aff_ref[1:2, :])

    @pl.when(s >= _NB)
    def _ip_phase():
        j = s - _NB
        tb = t_s[pl.ds(j * _BC, _BC), :]
        rec_ref[...] = jax.lax.dot_general(
            tb[:, :_H], t_s[:, :_H], (((1,), (1,)), ((), ())),
            preferred_element_type=jnp.float32)
        mu_ref[...] = tb[:, :_H]
        lv_ref[...] = tb[:, _H:]
        xrec_ref[...] = xrec_s[pl.ds(j * _BC, _BC), :]


def kernel(x, adj, W1, W2, Wfc, bfc, gamma, beta, running_mean, running_var):
    n, d = x.shape
    h = W1.shape[1]

    wcat = jnp.concatenate([W1, W2], axis=1)
    scale = gamma * jax.lax.rsqrt(running_var + 1e-5)
    shift = (bfc - running_mean) * scale + beta
    aff = jnp.stack([scale, shift], axis=0)

    xwt = pl.pallas_call(
        _xwt_kernel,
        grid=(1,),
        in_specs=[
            pl.BlockSpec((n, d), lambda i: (0, 0)),
            pl.BlockSpec((d, 2 * h), lambda i: (0, 0)),
        ],
        out_specs=pl.BlockSpec((2 * h, n), lambda i: (0, 0)),
        out_shape=jax.ShapeDtypeStruct((2 * h, n), jnp.float32),
    )(x, wcat)

    gc_idx = lambda s: (jnp.clip(s, 0, _NB - 1), 0)
    ip_idx = lambda s: (jnp.clip(s - _NB, 0, _NC - 1), 0)

    mu, logvar, x_rec, adj_rec = pl.pallas_call(
        _mega_kernel,
        grid=(_NB + _NC,),
        in_specs=[
            pl.BlockSpec((_BM, n), gc_idx),
            pl.BlockSpec((2 * h, n), lambda s: (0, 0)),
            pl.BlockSpec((h, d), lambda s: (0, 0)),
            pl.BlockSpec((2, d), lambda s: (0, 0)),
        ],
        out_specs=[
            pl.BlockSpec((_BC, h), ip_idx),
            pl.BlockSpec((_BC, h), ip_idx),
            pl.BlockSpec((_BC, d), ip_idx),
            pl.BlockSpec((_BC, n), ip_idx),
        ],
        out_shape=[
            jax.ShapeDtypeStruct((n, h), jnp.float32),
            jax.ShapeDtypeStruct((n, h), jnp.float32),
            jax.ShapeDtypeStruct((n, d), jnp.float32),
            jax.ShapeDtypeStruct((n, n), jnp.float32),
        ],
        scratch_shapes=[
            pltpu.VMEM((n, 2 * h), jnp.float32),
            pltpu.VMEM((n, d), jnp.float32),
        ],
    )(adj, xwt, Wfc, aff)

    z = mu
    return (adj_rec, mu, logvar, z, x_rec)


# speedup vs baseline: 1.0096x; 1.0053x over previous
import jax
import jax.numpy as jnp
from jax.experimental import pallas as pl
from jax.experimental.pallas import tpu as pltpu

_N, _D, _H = 10000, 128, 16
_BM = 200
_NB = _N // _BM
_BC = 200
_NC = _N // _BC


def _xwt_kernel(x_ref, w_ref, out_ref):
    out_ref[...] = jnp.dot(x_ref[...], w_ref[...],
                           preferred_element_type=jnp.float32).T


def _mega_kernel(adj_ref, xwt_ref, wfc_ref, aff_ref,
                 mu_ref, lv_ref, xrec_ref, rec_ref,
                 t_s):
    s = pl.program_id(0)

    @pl.when(s < _NB)
    def _gc_phase():
        t = jax.lax.dot_general(
            adj_ref[...], xwt_ref[...], (((1,), (1,)), ((), ())),
            preferred_element_type=jnp.float32)
        t = jnp.where(t >= 0, t, 0.01 * t)
        t_s[pl.ds(s * _BM, _BM), :] = t

    @pl.when(s >= _NB)
    def _ip_phase():
        j = s - _NB
        tb = t_s[pl.ds(j * _BC, _BC), :]
        rec_ref[...] = jax.lax.dot_general(
            tb[:, :_H], t_s[:, :_H], (((1,), (1,)), ((), ())),
            preferred_element_type=jnp.float32)
        mu_ref[...] = tb[:, :_H]
        lv_ref[...] = tb[:, _H:]
        h = jnp.dot(tb[:, :_H], wfc_ref[...],
                    preferred_element_type=jnp.float32)
        xrec_ref[...] = h * aff_ref[0:1, :] + aff_ref[1:2, :]


def kernel(x, adj, W1, W2, Wfc, bfc, gamma, beta, running_mean, running_var):
    n, d = x.shape
    h = W1.shape[1]

    wcat = jnp.concatenate([W1, W2], axis=1)
    scale = gamma * jax.lax.rsqrt(running_var + 1e-5)
    shift = (bfc - running_mean) * scale + beta
    aff = jnp.stack([scale, shift], axis=0)

    xwt = pl.pallas_call(
        _xwt_kernel,
        grid=(1,),
        in_specs=[
            pl.BlockSpec((n, d), lambda i: (0, 0)),
            pl.BlockSpec((d, 2 * h), lambda i: (0, 0)),
        ],
        out_specs=pl.BlockSpec((2 * h, n), lambda i: (0, 0)),
        out_shape=jax.ShapeDtypeStruct((2 * h, n), jnp.float32),
    )(x, wcat)

    gc_idx = lambda s: (jnp.clip(s, 0, _NB - 1), 0)
    ip_idx = lambda s: (jnp.clip(s - _NB, 0, _NC - 1), 0)

    mu, logvar, x_rec, adj_rec = pl.pallas_call(
        _mega_kernel,
        grid=(_NB + _NC,),
        in_specs=[
            pl.BlockSpec((_BM, n), gc_idx),
            pl.BlockSpec((2 * h, n), lambda s: (0, 0)),
            pl.BlockSpec((h, d), lambda s: (0, 0)),
            pl.BlockSpec((2, d), lambda s: (0, 0)),
        ],
        out_specs=[
            pl.BlockSpec((_BC, h), ip_idx),
            pl.BlockSpec((_BC, h), ip_idx),
            pl.BlockSpec((_BC, d), ip_idx),
            pl.BlockSpec((_BC, n), ip_idx),
        ],
        out_shape=[
            jax.ShapeDtypeStruct((n, h), jnp.float32),
            jax.ShapeDtypeStruct((n, h), jnp.float32),
            jax.ShapeDtypeStruct((n, d), jnp.float32),
            jax.ShapeDtypeStruct((n, n), jnp.float32),
        ],
        scratch_shapes=[
            pltpu.VMEM((n, 2 * h), jnp.float32),
        ],
    )(adj, xwt, Wfc, aff)

    z = mu
    return (adj_rec, mu, logvar, z, x_rec)
